# baseline (speedup 1.0000x reference)
import functools

import jax
import jax.numpy as jnp
from jax import lax
from jax.experimental import pallas as pl
from jax.experimental.pallas import tpu as pltpu
from jax.experimental.pallas import tpu_sc as plsc

M = 1000000
N = 1000000
D = 32
B = 16384
H1 = 64
H2 = 32

_info = plsc.get_sparse_core_info()
_NC, _NS = _info.num_cores, _info.num_subcores
_NW = _NC * _NS
_BPW = B // _NW

_sc_mesh = plsc.VectorSubcoreMesh(core_axis_name="c", subcore_axis_name="s")


@functools.partial(
    pl.kernel,
    mesh=_sc_mesh,
    out_type=[
        jax.ShapeDtypeStruct((B, D), jnp.float32),
        jax.ShapeDtypeStruct((B, D), jnp.float32),
    ],
    scratch_types=[
        pltpu.VMEM((_BPW,), jnp.int32),
        pltpu.VMEM((_BPW,), jnp.int32),
        pltpu.VMEM((_BPW, D), jnp.float32),
        pltpu.VMEM((_BPW, D), jnp.float32),
        pltpu.SemaphoreType.DMA,
        pltpu.SemaphoreType.DMA,
    ],
    compiler_params=pltpu.CompilerParams(use_tc_tiling_on_sc=False),
)
def _sc_gather(uid_hbm, bid_hbm, utab_hbm, btab_hbm, uout_hbm, bout_hbm,
               uidx_v, bidx_v, urows_v, brows_v, sem_u, sem_b):
    wid = lax.axis_index("s") * _NC + lax.axis_index("c")
    base = wid * _BPW
    pltpu.sync_copy(uid_hbm.at[pl.ds(base, _BPW)], uidx_v)
    pltpu.sync_copy(bid_hbm.at[pl.ds(base, _BPW)], bidx_v)
    cu = pltpu.async_copy(utab_hbm.at[uidx_v], urows_v, sem_u)
    cb = pltpu.async_copy(btab_hbm.at[bidx_v], brows_v, sem_b)
    cu.wait()
    cb.wait()
    pltpu.sync_copy(urows_v, uout_hbm.at[pl.ds(base, _BPW)])
    pltpu.sync_copy(brows_v, bout_hbm.at[pl.ds(base, _BPW)])


_BLK = 2048


def _mlp_body(u_ref, b_ref, w1u_ref, w1b_ref, b1_ref, w2_ref, b2_ref, o_ref):
    h = u_ref[...] @ w1u_ref[...] + b_ref[...] @ w1b_ref[...] + b1_ref[...]
    h = jnp.maximum(h, 0.0)
    o = h @ w2_ref[...] + b2_ref[...]
    o_ref[...] = jnp.maximum(o, 0.0)


def _mlp(u_emb, b_emb, w1u_t, w1b_t, b1_row, w2_t, b2_row):
    grid = (B // _BLK,)
    return pl.pallas_call(
        _mlp_body,
        grid=grid,
        in_specs=[
            pl.BlockSpec((_BLK, D), lambda i: (i, 0)),
            pl.BlockSpec((_BLK, D), lambda i: (i, 0)),
            pl.BlockSpec((D, H1), lambda i: (0, 0)),
            pl.BlockSpec((D, H1), lambda i: (0, 0)),
            pl.BlockSpec((1, H1), lambda i: (0, 0)),
            pl.BlockSpec((H1, H2), lambda i: (0, 0)),
            pl.BlockSpec((1, H2), lambda i: (0, 0)),
        ],
        out_specs=pl.BlockSpec((_BLK, H2), lambda i: (i, 0)),
        out_shape=jax.ShapeDtypeStruct((B, H2), jnp.float32),
    )(u_emb, b_emb, w1u_t, w1b_t, b1_row, w2_t, b2_row)


def kernel(user_id, book_id, user_table, book_table, W1, b1, W2, b2):
    uid = user_id.astype(jnp.int32)
    bid = book_id.astype(jnp.int32)
    u_emb, b_emb = _sc_gather(uid, bid, user_table, book_table)
    w1u_t = W1[:, :D].T
    w1b_t = W1[:, D:].T
    w2_t = W2.T
    return _mlp(u_emb, b_emb, w1u_t, w1b_t,
                b1.reshape(1, H1), w2_t, b2.reshape(1, H2))
